# baseline (speedup 1.0000x reference)
import jax
import jax.numpy as jnp
from jax import lax
from jax.experimental import pallas as pl
from jax.experimental.pallas import tpu as pltpu
from jax.experimental.pallas import tpu_sc as plsc

H = 32
W = 32
D = 128
BS = 16
NC = 2
NS = 16
L = 16
CPT = (2 * D) // NS


def _pos_body(row_hbm, col_hbm, out_hbm, tab, chunk, sem):
    cid = lax.axis_index("c")
    sid = lax.axis_index("s")

    pltpu.sync_copy(col_hbm.at[pl.ds(0, H * D)], tab.at[pl.ds(0, H * D)])
    pltpu.sync_copy(row_hbm.at[pl.ds(0, H * D)], tab.at[pl.ds(H * D, H * D)])

    iota = lax.iota(jnp.int32, L)

    for j in range(CPT):
        ch = sid * CPT + j
        is_col = ch < D
        iscolv = jnp.full((L,), is_col)
        cc = lax.rem(ch, D)
        ccv = jnp.full((L,), cc, dtype=jnp.int32)

        vcol0 = plsc.load_gather(tab, [iota * D + ccv])
        vcol1 = plsc.load_gather(tab, [(iota + L) * D + ccv])

        for y in range(H):
            vrow = plsc.load_gather(tab, [jnp.full((L,), (H + y) * D, jnp.int32) + ccv])
            v0 = jnp.where(iscolv, vcol0, vrow)
            v1 = jnp.where(iscolv, vcol1, vrow)
            chunk[j, y, pl.ds(0, L)] = v0
            chunk[j, y, pl.ds(L, L)] = v1

    copies = []
    for b in range(BS // NC):
        bb = cid * (BS // NC) + b
        copies.append(
            pltpu.async_copy(chunk, out_hbm.at[bb, pl.ds(sid * CPT, CPT)], sem)
        )
    for c in copies:
        c.wait()


@jax.jit
def _pos_embed(row_embed, col_embed):
    mesh = plsc.VectorSubcoreMesh(
        core_axis_name="c", subcore_axis_name="s", num_cores=NC, num_subcores=NS
    )
    return pl.kernel(
        _pos_body,
        out_type=jax.ShapeDtypeStruct((BS, 2 * D, H, W), jnp.float32),
        mesh=mesh,
        scratch_types=[
            pltpu.VMEM((2 * H * D,), jnp.float32),
            pltpu.VMEM((CPT, H, W), jnp.float32),
            pltpu.SemaphoreType.DMA,
        ],
        compiler_params=pltpu.CompilerParams(needs_layout_passes=False),
    )(row_embed.reshape(-1), col_embed.reshape(-1))


def kernel(mask, row_embed, col_embed):
    del mask
    return _pos_embed(row_embed, col_embed)


# speedup vs baseline: 1.0034x; 1.0034x over previous
import jax
import jax.numpy as jnp
from jax import lax
from jax.experimental import pallas as pl
from jax.experimental.pallas import tpu as pltpu
from jax.experimental.pallas import tpu_sc as plsc

H = 32
W = 32
D = 128
BS = 16
NC = 2
NS = 16
L = 16
CPT = (2 * D) // NS


def _pos_body(row_hbm, col_hbm, out_hbm, tab, chunk, sem):
    cid = lax.axis_index("c")
    sid = lax.axis_index("s")

    pltpu.sync_copy(col_hbm.at[pl.ds(0, H * D)], tab.at[pl.ds(0, H * D)])
    pltpu.sync_copy(row_hbm.at[pl.ds(0, H * D)], tab.at[pl.ds(H * D, H * D)])

    iota = lax.iota(jnp.int32, L)

    for j in range(CPT):
        ch = sid * CPT + j
        is_col = ch < D
        iscolv = jnp.full((L,), is_col)
        cc = lax.rem(ch, D)
        ccv = jnp.full((L,), cc, dtype=jnp.int32)

        vcol0 = plsc.load_gather(tab, [iota * D + ccv])
        vcol1 = plsc.load_gather(tab, [(iota + L) * D + ccv])

        for y in range(H):
            vrow = plsc.load_gather(tab, [jnp.full((L,), (H + y) * D, jnp.int32) + ccv])
            v0 = jnp.where(iscolv, vcol0, vrow)
            v1 = jnp.where(iscolv, vcol1, vrow)
            chunk[j, y, pl.ds(0, L)] = v0
            chunk[j, y, pl.ds(L, L)] = v1

    copies = []
    for b in range(BS // NC):
        bb = cid * (BS // NC) + b
        copies.append(
            pltpu.async_copy(chunk, out_hbm.at[bb, pl.ds(sid * CPT, CPT)], sem)
        )
    for c in copies:
        c.wait()


@jax.jit
def _pos_embed(row_embed, col_embed):
    mesh = plsc.VectorSubcoreMesh(
        core_axis_name="c", subcore_axis_name="s", num_cores=NC, num_subcores=NS
    )
    return pl.kernel(
        _pos_body,
        out_type=jax.ShapeDtypeStruct((BS, 2 * D, H, W), jnp.float32),
        mesh=mesh,
        scratch_types=[
            pltpu.VMEM((2 * H * D,), jnp.float32),
            pltpu.VMEM((CPT, H, W), jnp.float32),
            pltpu.SemaphoreType.DMA,
        ],
        compiler_params=pltpu.CompilerParams(
            needs_layout_passes=False, use_tc_tiling_on_sc=True
        ),
    )(row_embed.reshape(-1), col_embed.reshape(-1))


def kernel(mask, row_embed, col_embed):
    del mask
    return _pos_embed(row_embed, col_embed)


# speedup vs baseline: 1.3759x; 1.3713x over previous
import jax
import jax.numpy as jnp
from jax.experimental import pallas as pl
from jax.experimental.pallas import tpu as pltpu

H = 32
W = 32
D = 128
BS = 16


def _body(row_ref, col_ref, out_ref, img):
    b = pl.program_id(0)

    @pl.when(b == 0)
    def _():
        colT = col_ref[...].T
        rowT = row_ref[...].T
        xe = jnp.broadcast_to(colT[:, None, :], (D, H, W))
        ye = jnp.broadcast_to(rowT[:, :, None], (D, H, W))
        img[...] = jnp.concatenate([xe, ye], axis=0)

    out_ref[0] = img[...]


@jax.jit
def _pos_embed(row_embed, col_embed):
    return pl.pallas_call(
        _body,
        grid=(BS,),
        in_specs=[
            pl.BlockSpec((H, D), lambda b: (0, 0)),
            pl.BlockSpec((H, D), lambda b: (0, 0)),
        ],
        out_specs=pl.BlockSpec((1, 2 * D, H, W), lambda b: (b, 0, 0, 0)),
        out_shape=jax.ShapeDtypeStruct((BS, 2 * D, H, W), jnp.float32),
        scratch_shapes=[pltpu.VMEM((2 * D, H, W), jnp.float32)],
    )(row_embed[:H], col_embed[:H])


def kernel(mask, row_embed, col_embed):
    del mask
    return _pos_embed(row_embed, col_embed)
